# baseline (speedup 1.0000x reference)
import functools

import jax
import jax.numpy as jnp
from jax import lax
from jax.experimental import pallas as pl
from jax.experimental.pallas import tpu as pltpu
from jax.experimental.pallas import tpu_sc as plsc

NUM_CORES = 2
NUM_SUBCORES = 16
NUM_WORKERS = NUM_CORES * NUM_SUBCORES
BATCH = 16384
BPW = BATCH // NUM_WORKERS
CHUNK = 128
NCHUNK = BPW // CHUNK
D = 5
LANES = 16


def _partials_body(p_hbm, q_hbm, uid_hbm, iid_hbm, out_hbm,
                   uid_v, iid_v, p_v, q_v, acc_v,
                   sem_p, sem_q):
    wid = lax.axis_index("s") * NUM_CORES + lax.axis_index("c")
    pltpu.sync_copy(uid_hbm.at[wid], uid_v)
    pltpu.sync_copy(iid_hbm.at[wid], iid_v)
    copies = []
    for k in range(NCHUNK):
        dst = pl.ds(k * CHUNK, CHUNK)
        copies.append(pltpu.async_copy(p_hbm.at[uid_v.at[k]], p_v.at[dst], sem_p))
        copies.append(pltpu.async_copy(q_hbm.at[iid_v.at[k]], q_v.at[dst], sem_q))
    for cp in copies:
        cp.wait()
    iota = lax.iota(jnp.int32, LANES)
    acc = jnp.zeros((LANES,), jnp.float32)
    for j in range(BPW // LANES):
        rows = j * LANES + iota
        for c in range(D):
            cols = jnp.full((LANES,), c, jnp.int32)
            pv = plsc.load_gather(p_v, [rows, cols])
            qv = plsc.load_gather(q_v, [rows, cols])
            acc = acc + pv * qv
    acc_v[...] = acc
    pltpu.sync_copy(acc_v, out_hbm.at[wid])


_lfm_partials = functools.partial(
    pl.kernel,
    out_type=jax.ShapeDtypeStruct((NUM_WORKERS, LANES), jnp.float32),
    mesh=plsc.VectorSubcoreMesh(core_axis_name="c", subcore_axis_name="s",
                                num_cores=NUM_CORES, num_subcores=NUM_SUBCORES),
    compiler_params=pltpu.CompilerParams(needs_layout_passes=False,
                                         use_tc_tiling_on_sc=False),
    scratch_types=[
        pltpu.VMEM((NCHUNK, CHUNK), jnp.int32),
        pltpu.VMEM((NCHUNK, CHUNK), jnp.int32),
        pltpu.VMEM((BPW, D), jnp.float32),
        pltpu.VMEM((BPW, D), jnp.float32),
        pltpu.VMEM((LANES,), jnp.float32),
        pltpu.SemaphoreType.DMA,
        pltpu.SemaphoreType.DMA,
    ],
)(_partials_body)


def _finish_body(x_ref, o_ref):
    r = jnp.sum(x_ref[...])
    o_ref[0, 0] = 1.0 / (1.0 + jnp.exp(-r))


def kernel(P, Q, user_id, item_id):
    uid = user_id.astype(jnp.int32).reshape(NUM_WORKERS, NCHUNK, CHUNK)
    iid = item_id.astype(jnp.int32).reshape(NUM_WORKERS, NCHUNK, CHUNK)
    partials = _lfm_partials(P, Q, uid, iid)
    out = pl.pallas_call(
        _finish_body,
        out_shape=jax.ShapeDtypeStruct((1, 1), jnp.float32),
        out_specs=pl.BlockSpec(memory_space=pltpu.SMEM),
    )(partials.reshape(NUM_WORKERS // 8, 8 * LANES))
    return out[0, 0]


# speedup vs baseline: 1.0051x; 1.0051x over previous
import functools

import jax
import jax.numpy as jnp
from jax import lax
from jax.experimental import pallas as pl
from jax.experimental.pallas import tpu as pltpu
from jax.experimental.pallas import tpu_sc as plsc

NUM_CORES = 2
NUM_SUBCORES = 16
NUM_WORKERS = NUM_CORES * NUM_SUBCORES
BATCH = 16384
BPW = BATCH // NUM_WORKERS
CHUNK = 128
NCHUNK = BPW // CHUNK
D = 5
DPAD = 8
LANES = 16


def _partials_body(p_hbm, q_hbm, uid_hbm, iid_hbm, out_hbm,
                   uid_v, iid_v, p_v, q_v, acc_v,
                   sem_p, sem_q):
    wid = lax.axis_index("s") * NUM_CORES + lax.axis_index("c")
    pltpu.sync_copy(uid_hbm.at[wid], uid_v)
    pltpu.sync_copy(iid_hbm.at[wid], iid_v)
    copies = []
    for k in range(NCHUNK):
        dst = pl.ds(k * CHUNK, CHUNK)
        copies.append(pltpu.async_copy(p_hbm.at[uid_v.at[k]], p_v.at[dst], sem_p))
        copies.append(pltpu.async_copy(q_hbm.at[iid_v.at[k]], q_v.at[dst], sem_q))
    for cp in copies:
        cp.wait()
    iota = lax.iota(jnp.int32, LANES)
    rows_base = iota // DPAD
    cols = iota % DPAD
    acc = jnp.zeros((LANES,), jnp.float32)
    for j in range(BPW * DPAD // LANES):
        rows = rows_base + (j * LANES // DPAD)
        pv = plsc.load_gather(p_v, [rows, cols])
        qv = plsc.load_gather(q_v, [rows, cols])
        acc = acc + pv * qv
    acc_v[...] = acc
    pltpu.sync_copy(acc_v, out_hbm.at[wid])


_lfm_partials = functools.partial(
    pl.kernel,
    out_type=jax.ShapeDtypeStruct((NUM_WORKERS, LANES), jnp.float32),
    mesh=plsc.VectorSubcoreMesh(core_axis_name="c", subcore_axis_name="s",
                                num_cores=NUM_CORES, num_subcores=NUM_SUBCORES),
    compiler_params=pltpu.CompilerParams(needs_layout_passes=False,
                                         use_tc_tiling_on_sc=False),
    scratch_types=[
        pltpu.VMEM((NCHUNK, CHUNK), jnp.int32),
        pltpu.VMEM((NCHUNK, CHUNK), jnp.int32),
        pltpu.VMEM((BPW, DPAD), jnp.float32),
        pltpu.VMEM((BPW, DPAD), jnp.float32),
        pltpu.VMEM((LANES,), jnp.float32),
        pltpu.SemaphoreType.DMA,
        pltpu.SemaphoreType.DMA,
    ],
)(_partials_body)


def _finish_body(x_ref, o_ref):
    r = jnp.sum(x_ref[...])
    o_ref[0, 0] = 1.0 / (1.0 + jnp.exp(-r))


def kernel(P, Q, user_id, item_id):
    Pp = jnp.pad(P, ((0, 0), (0, DPAD - D)))
    Qp = jnp.pad(Q, ((0, 0), (0, DPAD - D)))
    uid = user_id.astype(jnp.int32).reshape(NUM_WORKERS, NCHUNK, CHUNK)
    iid = item_id.astype(jnp.int32).reshape(NUM_WORKERS, NCHUNK, CHUNK)
    partials = _lfm_partials(Pp, Qp, uid, iid)
    out = pl.pallas_call(
        _finish_body,
        out_shape=jax.ShapeDtypeStruct((1, 1), jnp.float32),
        out_specs=pl.BlockSpec(memory_space=pltpu.SMEM),
    )(partials.reshape(NUM_WORKERS // 8, 8 * LANES))
    return out[0, 0]


# speedup vs baseline: 10.3745x; 10.3219x over previous
import functools

import jax
import jax.numpy as jnp
from jax import lax
from jax.experimental import pallas as pl
from jax.experimental.pallas import tpu as pltpu
from jax.experimental.pallas import tpu_sc as plsc

NUM_CORES = 2
NUM_SUBCORES = 16
NUM_WORKERS = NUM_CORES * NUM_SUBCORES
BATCH = 16384
BPW = BATCH // NUM_WORKERS
G = 16
NGROUPS = BPW // G
DEPTH = 2
D = 5
LANES = 16
W = 128
STRIP = G * W


def _issue_group(pt_hbm, qt_hbm, uid_v, iid_v, pbuf, qbuf, sem_p, sem_q, g):
    d = g % DEPTH
    uvec = uid_v[pl.ds(g * G, G)]
    ivec = iid_v[pl.ds(g * G, G)]
    ubase = (uvec // W) * W
    ibase = (ivec // W) * W
    for k in range(G):
        su = pl.multiple_of(ubase[k], W)
        si = pl.multiple_of(ibase[k], W)
        dst = pl.ds(d * STRIP + k * W, W)
        pltpu.async_copy(pt_hbm.at[:, pl.ds(su, W)], pbuf.at[:, dst], sem_p)
        pltpu.async_copy(qt_hbm.at[:, pl.ds(si, W)], qbuf.at[:, dst], sem_q)


def _drain_group(pt_hbm, qt_hbm, pbuf, qbuf, sem_p, sem_q, g):
    d = g % DEPTH
    strip = pl.ds(d * STRIP, STRIP)
    pltpu.make_async_copy(pt_hbm.at[:, pl.ds(0, STRIP)], pbuf.at[:, strip],
                          sem_p).wait()
    pltpu.make_async_copy(qt_hbm.at[:, pl.ds(0, STRIP)], qbuf.at[:, strip],
                          sem_q).wait()


def _process_group(uid_v, iid_v, pbuf, qbuf, g, acc):
    d = g % DEPTH
    iota = lax.iota(jnp.int32, LANES)
    uvec = uid_v[pl.ds(g * G, G)]
    ivec = iid_v[pl.ds(g * G, G)]
    uoff = d * STRIP + iota * W + lax.rem(uvec, jnp.int32(W))
    ioff = d * STRIP + iota * W + lax.rem(ivec, jnp.int32(W))
    for c in range(D):
        cs = jnp.full((LANES,), c, jnp.int32)
        pv = plsc.load_gather(pbuf, [cs, uoff])
        qv = plsc.load_gather(qbuf, [cs, ioff])
        acc = acc + pv * qv
    return acc


def _partials_body(pt_hbm, qt_hbm, uid_hbm, iid_hbm, out_hbm,
                   uid_v, iid_v, pbuf, qbuf, acc_v, sem_p, sem_q):
    wid = lax.axis_index("s") * NUM_CORES + lax.axis_index("c")
    pltpu.sync_copy(uid_hbm.at[pl.ds(wid * BPW, BPW)], uid_v)
    pltpu.sync_copy(iid_hbm.at[pl.ds(wid * BPW, BPW)], iid_v)

    for g in range(DEPTH):
        _issue_group(pt_hbm, qt_hbm, uid_v, iid_v, pbuf, qbuf, sem_p, sem_q, g)

    acc = jnp.zeros((LANES,), jnp.float32)
    for g in range(NGROUPS):
        _drain_group(pt_hbm, qt_hbm, pbuf, qbuf, sem_p, sem_q, g)
        acc = _process_group(uid_v, iid_v, pbuf, qbuf, g, acc)
        if g + DEPTH < NGROUPS:
            _issue_group(pt_hbm, qt_hbm, uid_v, iid_v, pbuf, qbuf,
                         sem_p, sem_q, g + DEPTH)

    acc_v[...] = acc
    pltpu.sync_copy(acc_v, out_hbm.at[wid])


_lfm_partials = functools.partial(
    pl.kernel,
    out_type=jax.ShapeDtypeStruct((NUM_WORKERS, LANES), jnp.float32),
    mesh=plsc.VectorSubcoreMesh(core_axis_name="c", subcore_axis_name="s",
                                num_cores=NUM_CORES, num_subcores=NUM_SUBCORES),
    compiler_params=pltpu.CompilerParams(needs_layout_passes=False,
                                         use_tc_tiling_on_sc=True),
    scratch_types=[
        pltpu.VMEM((BPW,), jnp.int32),
        pltpu.VMEM((BPW,), jnp.int32),
        pltpu.VMEM((D, DEPTH * STRIP), jnp.float32),
        pltpu.VMEM((D, DEPTH * STRIP), jnp.float32),
        pltpu.VMEM((LANES,), jnp.float32),
        pltpu.SemaphoreType.DMA,
        pltpu.SemaphoreType.DMA,
    ],
)(_partials_body)


def _finish_body(x_ref, o_ref):
    r = jnp.sum(x_ref[...])
    o_ref[0, 0] = 1.0 / (1.0 + jnp.exp(-r))


def kernel(P, Q, user_id, item_id):
    partials = _lfm_partials(P.T, Q.T,
                             user_id.astype(jnp.int32),
                             item_id.astype(jnp.int32))
    out = pl.pallas_call(
        _finish_body,
        out_shape=jax.ShapeDtypeStruct((1, 1), jnp.float32),
        out_specs=pl.BlockSpec(memory_space=pltpu.SMEM),
    )(partials)
    return out[0, 0]


# speedup vs baseline: 10.4196x; 1.0044x over previous
import functools

import jax
import jax.numpy as jnp
from jax import lax
from jax.experimental import pallas as pl
from jax.experimental.pallas import tpu as pltpu
from jax.experimental.pallas import tpu_sc as plsc

NUM_CORES = 2
NUM_SUBCORES = 16
NUM_WORKERS = NUM_CORES * NUM_SUBCORES
BATCH = 16384
BPW = BATCH // NUM_WORKERS
G = 16
NGROUPS = BPW // G
DEPTH = 3
D = 5
LANES = 16
W = 128
STRIP = G * W


def _issue_group(pt_hbm, qt_hbm, uid_v, iid_v, pbuf, qbuf, sem_p, sem_q, g):
    d = g % DEPTH
    uvec = uid_v[pl.ds(g * G, G)]
    ivec = iid_v[pl.ds(g * G, G)]
    ubase = (uvec // W) * W
    ibase = (ivec // W) * W
    for k in range(G):
        su = pl.multiple_of(ubase[k], W)
        si = pl.multiple_of(ibase[k], W)
        dst = pl.ds(d * STRIP + k * W, W)
        pltpu.async_copy(pt_hbm.at[:, pl.ds(su, W)], pbuf.at[:, dst], sem_p)
        pltpu.async_copy(qt_hbm.at[:, pl.ds(si, W)], qbuf.at[:, dst], sem_q)


def _drain_group(pt_hbm, qt_hbm, pbuf, qbuf, sem_p, sem_q, g):
    d = g % DEPTH
    strip = pl.ds(d * STRIP, STRIP)
    pltpu.make_async_copy(pt_hbm.at[:, pl.ds(0, STRIP)], pbuf.at[:, strip],
                          sem_p).wait()
    pltpu.make_async_copy(qt_hbm.at[:, pl.ds(0, STRIP)], qbuf.at[:, strip],
                          sem_q).wait()


def _process_group(uid_v, iid_v, pbuf, qbuf, g, acc):
    d = g % DEPTH
    iota = lax.iota(jnp.int32, LANES)
    uvec = uid_v[pl.ds(g * G, G)]
    ivec = iid_v[pl.ds(g * G, G)]
    uoff = d * STRIP + iota * W + lax.rem(uvec, jnp.int32(W))
    ioff = d * STRIP + iota * W + lax.rem(ivec, jnp.int32(W))
    for c in range(D):
        cs = jnp.full((LANES,), c, jnp.int32)
        pv = plsc.load_gather(pbuf, [cs, uoff])
        qv = plsc.load_gather(qbuf, [cs, ioff])
        acc = acc + pv * qv
    return acc


def _partials_body(pt_hbm, qt_hbm, uid_hbm, iid_hbm, out_hbm,
                   uid_v, iid_v, pbuf, qbuf, acc_v, sem_p, sem_q):
    wid = lax.axis_index("s") * NUM_CORES + lax.axis_index("c")
    pltpu.sync_copy(uid_hbm.at[pl.ds(wid * BPW, BPW)], uid_v)
    pltpu.sync_copy(iid_hbm.at[pl.ds(wid * BPW, BPW)], iid_v)

    for g in range(DEPTH):
        _issue_group(pt_hbm, qt_hbm, uid_v, iid_v, pbuf, qbuf, sem_p, sem_q, g)

    acc = jnp.zeros((LANES,), jnp.float32)
    for g in range(NGROUPS):
        _drain_group(pt_hbm, qt_hbm, pbuf, qbuf, sem_p, sem_q, g)
        acc = _process_group(uid_v, iid_v, pbuf, qbuf, g, acc)
        if g + DEPTH < NGROUPS:
            _issue_group(pt_hbm, qt_hbm, uid_v, iid_v, pbuf, qbuf,
                         sem_p, sem_q, g + DEPTH)

    acc_v[...] = acc
    pltpu.sync_copy(acc_v, out_hbm.at[wid])


_lfm_partials = functools.partial(
    pl.kernel,
    out_type=jax.ShapeDtypeStruct((NUM_WORKERS, LANES), jnp.float32),
    mesh=plsc.VectorSubcoreMesh(core_axis_name="c", subcore_axis_name="s",
                                num_cores=NUM_CORES, num_subcores=NUM_SUBCORES),
    compiler_params=pltpu.CompilerParams(needs_layout_passes=False,
                                         use_tc_tiling_on_sc=True),
    scratch_types=[
        pltpu.VMEM((BPW,), jnp.int32),
        pltpu.VMEM((BPW,), jnp.int32),
        pltpu.VMEM((D, DEPTH * STRIP), jnp.float32),
        pltpu.VMEM((D, DEPTH * STRIP), jnp.float32),
        pltpu.VMEM((LANES,), jnp.float32),
        pltpu.SemaphoreType.DMA,
        pltpu.SemaphoreType.DMA,
    ],
)(_partials_body)


def _finish_body(x_ref, o_ref):
    r = jnp.sum(x_ref[...])
    o_ref[0, 0] = 1.0 / (1.0 + jnp.exp(-r))


def kernel(P, Q, user_id, item_id):
    partials = _lfm_partials(P.T, Q.T,
                             user_id.astype(jnp.int32),
                             item_id.astype(jnp.int32))
    out = pl.pallas_call(
        _finish_body,
        out_shape=jax.ShapeDtypeStruct((1, 1), jnp.float32),
        out_specs=pl.BlockSpec(memory_space=pltpu.SMEM),
    )(partials)
    return out[0, 0]


# speedup vs baseline: 10.5097x; 1.0086x over previous
import functools

import jax
import jax.numpy as jnp
from jax import lax
from jax.experimental import pallas as pl
from jax.experimental.pallas import tpu as pltpu
from jax.experimental.pallas import tpu_sc as plsc

NUM_CORES = 2
NUM_SUBCORES = 16
NUM_WORKERS = NUM_CORES * NUM_SUBCORES
BATCH = 16384
BPW = BATCH // NUM_WORKERS
G = 16
NGROUPS = BPW // G
DEPTH = 3
D = 5
LANES = 16
W = 128
STRIP = G * W


def _issue_group(pt_hbm, qt_hbm, uid_v, iid_v, pbuf, qbuf, sem_p, sem_q, g):
    d = g % DEPTH
    uvec = uid_v[pl.ds(g * G, G)]
    ivec = iid_v[pl.ds(g * G, G)]
    ubase = (uvec // W) * W
    ibase = (ivec // W) * W
    for k in range(G):
        su = pl.multiple_of(ubase[k], W)
        si = pl.multiple_of(ibase[k], W)
        dst = pl.ds(d * STRIP + k * W, W)
        pltpu.async_copy(pt_hbm.at[:, pl.ds(su, W)], pbuf.at[:, dst], sem_p)
        pltpu.async_copy(qt_hbm.at[:, pl.ds(si, W)], qbuf.at[:, dst], sem_q)


def _drain_group(pt_hbm, qt_hbm, pbuf, qbuf, sem_p, sem_q, g):
    d = g % DEPTH
    strip = pl.ds(d * STRIP, STRIP)
    pltpu.make_async_copy(pt_hbm.at[:, pl.ds(0, STRIP)], pbuf.at[:, strip],
                          sem_p).wait()
    pltpu.make_async_copy(qt_hbm.at[:, pl.ds(0, STRIP)], qbuf.at[:, strip],
                          sem_q).wait()


def _process_group(uid_v, iid_v, pbuf, qbuf, g, acc):
    d = g % DEPTH
    iota = lax.iota(jnp.int32, LANES)
    uvec = uid_v[pl.ds(g * G, G)]
    ivec = iid_v[pl.ds(g * G, G)]
    uoff = d * STRIP + iota * W + lax.rem(uvec, jnp.int32(W))
    ioff = d * STRIP + iota * W + lax.rem(ivec, jnp.int32(W))
    for c in range(D):
        cs = jnp.full((LANES,), c, jnp.int32)
        pv = plsc.load_gather(pbuf, [cs, uoff])
        qv = plsc.load_gather(qbuf, [cs, ioff])
        acc = acc + pv * qv
    return acc


def _partials_body(pt_hbm, qt_hbm, uid_hbm, iid_hbm, out_hbm,
                   uid_v, iid_v, pbuf, qbuf, acc_v, sem_p, sem_q):
    wid = lax.axis_index("s") * NUM_CORES + lax.axis_index("c")
    pltpu.sync_copy(uid_hbm.at[pl.ds(wid * BPW, BPW)], uid_v)
    pltpu.sync_copy(iid_hbm.at[pl.ds(wid * BPW, BPW)], iid_v)

    for g in range(DEPTH):
        _issue_group(pt_hbm, qt_hbm, uid_v, iid_v, pbuf, qbuf, sem_p, sem_q, g)

    acc = jnp.zeros((LANES,), jnp.float32)
    for g in range(NGROUPS):
        _drain_group(pt_hbm, qt_hbm, pbuf, qbuf, sem_p, sem_q, g)
        acc = _process_group(uid_v, iid_v, pbuf, qbuf, g, acc)
        if g + DEPTH < NGROUPS:
            _issue_group(pt_hbm, qt_hbm, uid_v, iid_v, pbuf, qbuf,
                         sem_p, sem_q, g + DEPTH)

    acc_v[...] = acc
    pltpu.sync_copy(acc_v, out_hbm.at[wid])


_lfm_partials = functools.partial(
    pl.kernel,
    out_type=jax.ShapeDtypeStruct((NUM_WORKERS, LANES), jnp.float32),
    mesh=plsc.VectorSubcoreMesh(core_axis_name="c", subcore_axis_name="s",
                                num_cores=NUM_CORES, num_subcores=NUM_SUBCORES),
    compiler_params=pltpu.CompilerParams(needs_layout_passes=False,
                                         use_tc_tiling_on_sc=True),
    scratch_types=[
        pltpu.VMEM((BPW,), jnp.int32),
        pltpu.VMEM((BPW,), jnp.int32),
        pltpu.VMEM((8, DEPTH * STRIP), jnp.float32),
        pltpu.VMEM((D, DEPTH * STRIP), jnp.float32),
        pltpu.VMEM((LANES,), jnp.float32),
        pltpu.SemaphoreType.DMA,
        pltpu.SemaphoreType.DMA,
    ],
)(_partials_body)


def _finish_body(x_ref, o_ref):
    r = jnp.sum(x_ref[...])
    o_ref[0, 0] = 1.0 / (1.0 + jnp.exp(-r))


def kernel(P, Q, user_id, item_id):
    PT8 = jnp.pad(P.T, ((0, 3), (0, 0)))
    partials = _lfm_partials(PT8, Q.T,
                             user_id.astype(jnp.int32),
                             item_id.astype(jnp.int32))
    out = pl.pallas_call(
        _finish_body,
        out_shape=jax.ShapeDtypeStruct((1, 1), jnp.float32),
        out_specs=pl.BlockSpec(memory_space=pltpu.SMEM),
    )(partials)
    return out[0, 0]
